# baseline (speedup 1.0000x reference)
import functools
import math

import jax
import jax.numpy as jnp
from jax import lax
from jax.experimental import pallas as pl
from jax.experimental.pallas import tpu as pltpu
from jax.experimental.pallas import tpu_sc as plsc

D_MODEL = 64
VOCAB = 1000000
ROWS = 4096
COLS = 200
B = ROWS * COLS
CH = 128
NW = 32
NCHUNKS = B // CH
CPW = NCHUNKS // NW
SCALE = math.sqrt(D_MODEL)
LANES = 16

_mesh = plsc.VectorSubcoreMesh(core_axis_name="c", subcore_axis_name="s")


@functools.partial(
    pl.kernel,
    mesh=_mesh,
    compiler_params=pltpu.CompilerParams(use_tc_tiling_on_sc=False),
    out_type=jax.ShapeDtypeStruct((NCHUNKS, CH, D_MODEL), jnp.float32),
    scratch_types=[
        pltpu.VMEM((CPW, CH), jnp.int32),
        pltpu.VMEM((CH, D_MODEL), jnp.float32),
        pltpu.SemaphoreType.DMA,
    ],
)
def _emb_lookup(lut_hbm, idx_hbm, out_hbm, idx_v, rows_v, gsem):
    wid = lax.axis_index("s") * 2 + lax.axis_index("c")
    base = wid * CPW
    pltpu.sync_copy(idx_hbm.at[pl.ds(base, CPW)], idx_v)

    def body(j, carry):
        pltpu.async_copy(lut_hbm.at[idx_v.at[j]], rows_v, gsem).wait()

        def scale_row(r, c):
            for t in range(D_MODEL // LANES):
                sl = pl.ds(t * LANES, LANES)
                rows_v[r, sl] = rows_v[r, sl] * SCALE
            return c

        lax.fori_loop(0, CH, scale_row, 0)
        pltpu.sync_copy(rows_v, out_hbm.at[base + j])
        return carry

    lax.fori_loop(0, CPW, body, 0)


def kernel(x, lut):
    idx = x.reshape(NCHUNKS, CH).astype(jnp.int32)
    out = _emb_lookup(lut, idx)
    return out.reshape(ROWS, COLS, D_MODEL)


# speedup vs baseline: 1.2060x; 1.2060x over previous
import functools
import math

import jax
import jax.numpy as jnp
from jax import lax
from jax.experimental import pallas as pl
from jax.experimental.pallas import tpu as pltpu
from jax.experimental.pallas import tpu_sc as plsc

D_MODEL = 64
ROWS = 4096
COLS = 200
B = ROWS * COLS
IDXW = 128
CHB = 256
NW = 32
NIDX = B // IDXW
OUT_ROWS = B // CHB
NB = OUT_ROWS // NW
IPW = NIDX // NW
SCALE = math.sqrt(D_MODEL)
LANES = 16

_mesh = plsc.VectorSubcoreMesh(core_axis_name="c", subcore_axis_name="s")


@functools.partial(
    pl.kernel,
    mesh=_mesh,
    compiler_params=pltpu.CompilerParams(use_tc_tiling_on_sc=False),
    out_type=jax.ShapeDtypeStruct((OUT_ROWS, CHB, D_MODEL), jnp.float32),
    scratch_types=[
        pltpu.VMEM((IPW, IDXW), jnp.int32),
        pltpu.VMEM((CHB, D_MODEL), jnp.float32),
        pltpu.VMEM((CHB, D_MODEL), jnp.float32),
        pltpu.VMEM((CHB, D_MODEL), jnp.float32),
        pltpu.VMEM((CHB, D_MODEL), jnp.float32),
        pltpu.SemaphoreType.DMA,
        pltpu.SemaphoreType.DMA,
        pltpu.SemaphoreType.DMA,
        pltpu.SemaphoreType.DMA,
    ],
)
def _emb_lookup(lut_hbm, idx_hbm, out_hbm, idx_v, gb0, gb1, wb0, wb1,
                gs0, gs1, ws0, ws1):
    wid = lax.axis_index("s") * 2 + lax.axis_index("c")
    ibase = wid * IPW
    obase = wid * NB
    gbufs, wbufs = (gb0, gb1), (wb0, wb1)
    gsems, wsems = (gs0, gs1), (ws0, ws1)

    pltpu.sync_copy(idx_hbm.at[pl.ds(ibase, IPW)], idx_v)

    def start_gathers(i, gb, gs):
        pltpu.async_copy(lut_hbm.at[idx_v.at[2 * i]],
                         gb.at[pl.ds(0, IDXW)], gs)
        pltpu.async_copy(lut_hbm.at[idx_v.at[2 * i + 1]],
                         gb.at[pl.ds(IDXW, IDXW)], gs)

    def wait_gathers(i, gb, gs):
        pltpu.make_async_copy(lut_hbm.at[idx_v.at[2 * i]],
                              gb.at[pl.ds(0, IDXW)], gs).wait()
        pltpu.make_async_copy(lut_hbm.at[idx_v.at[2 * i + 1]],
                              gb.at[pl.ds(IDXW, IDXW)], gs).wait()

    for b in range(2):
        start_gathers(b, gbufs[b], gsems[b])

    def body(jj, carry):
        for b in range(2):
            i = 2 * jj + b
            gb, wb = gbufs[b], wbufs[b]
            wait_gathers(i, gb, gsems[b])

            @pl.when(i >= 2)
            def _():
                pltpu.make_async_copy(wb, out_hbm.at[obase + i - 2],
                                      wsems[b]).wait()

            @plsc.parallel_loop(0, CHB, 1, unroll=2)
            def _(r):
                for t in range(D_MODEL // LANES):
                    sl = pl.ds(t * LANES, LANES)
                    wb[r, sl] = gb[r, sl] * SCALE

            @pl.when(i + 2 < NB)
            def _():
                start_gathers(i + 2, gb, gsems[b])

            pltpu.async_copy(wb, out_hbm.at[obase + i], wsems[b])
        return carry

    lax.fori_loop(0, NB // 2, body, 0)

    for b in range(2):
        pltpu.make_async_copy(wbufs[b], out_hbm.at[obase + NB - 2 + b],
                              wsems[b]).wait()


def kernel(x, lut):
    idx = x.reshape(NIDX, IDXW).astype(jnp.int32)
    out = _emb_lookup(lut, idx)
    return out.reshape(ROWS, COLS, D_MODEL)
